# baseline (speedup 1.0000x reference)
import functools

import jax
import jax.numpy as jnp
from jax import lax
from jax.experimental import pallas as pl
from jax.experimental.pallas import tpu as pltpu
from jax.experimental.pallas import tpu_sc as plsc

B = 16384
D = 64
NC = 2
NS = 16
NW = NC * NS
BPW = B // NW
L = 16
CHUNK = 128
NCHUNK = BPW // CHUNK


def _sc_scores(u_pos, v, u_table, v_table):
    mesh = plsc.VectorSubcoreMesh(core_axis_name="c", subcore_axis_name="s")

    @functools.partial(
        pl.kernel,
        out_type=jax.ShapeDtypeStruct((B,), jnp.float32),
        mesh=mesh,
        compiler_params=pltpu.CompilerParams(
            needs_layout_passes=False, use_tc_tiling_on_sc=False),
        scratch_types=[
            pltpu.VMEM((NCHUNK, CHUNK), jnp.int32),
            pltpu.VMEM((NCHUNK, CHUNK), jnp.int32),
            pltpu.VMEM((BPW, D), jnp.float32),
            pltpu.VMEM((BPW, D), jnp.float32),
            pltpu.VMEM((BPW,), jnp.float32),
            pltpu.VMEM((L * L,), jnp.float32),
            pltpu.SemaphoreType.DMA,
            pltpu.SemaphoreType.DMA,
        ],
    )
    def k(u_pos_hbm, v_hbm, u_table_hbm, v_table_hbm, out_hbm,
          uidx_v, vidx_v, urows_v, vrows_v, out_v, tbuf_v, sem_u, sem_v):
        wid = lax.axis_index("s") * NC + lax.axis_index("c")
        base = wid * BPW

        for i in range(NCHUNK):
            pltpu.sync_copy(u_pos_hbm.at[pl.ds(base + i * CHUNK, CHUNK)],
                            uidx_v.at[i])
            pltpu.sync_copy(v_hbm.at[pl.ds(base + i * CHUNK, CHUNK)],
                            vidx_v.at[i])

        urows_2d = urows_v
        vrows_2d = vrows_v
        for i in range(NCHUNK):
            pltpu.async_copy(u_table_hbm.at[uidx_v.at[i]],
                             urows_2d.at[pl.ds(i * CHUNK, CHUNK)], sem_u)
            pltpu.async_copy(v_table_hbm.at[vidx_v.at[i]],
                             vrows_2d.at[pl.ds(i * CHUNK, CHUNK)], sem_v)
        for i in range(NCHUNK):
            pltpu.make_async_copy(u_table_hbm.at[uidx_v.at[i]],
                                  urows_2d.at[pl.ds(i * CHUNK, CHUNK)],
                                  sem_u).wait()
            pltpu.make_async_copy(v_table_hbm.at[vidx_v.at[i]],
                                  vrows_2d.at[pl.ds(i * CHUNK, CHUNK)],
                                  sem_v).wait()

        lanes = lax.iota(jnp.int32, L)

        def group(g, carry):
            base_r = g * L
            for r in range(L):
                s = jnp.zeros((L,), jnp.float32)
                for j in range(D // L):
                    uu = urows_v[base_r + r, pl.ds(j * L, L)]
                    vv = vrows_v[base_r + r, pl.ds(j * L, L)]
                    s = s + uu * vv
                plsc.store_scatter(tbuf_v, [lanes * L + r], s)
            acc = jnp.zeros((L,), jnp.float32)
            for kk in range(L):
                acc = acc + tbuf_v[pl.ds(kk * L, L)]
            out_v[pl.ds(g * L, L)] = acc
            return carry

        lax.fori_loop(0, BPW // L, group, 0)
        pltpu.sync_copy(out_v, out_hbm.at[pl.ds(base, BPW)])

    return k(u_pos, v, u_table, v_table)


def _logsigmoid_tc(scores):
    x = scores.reshape(B // 128, 128)

    def body(x_ref, o_ref):
        o_ref[...] = jax.nn.log_sigmoid(x_ref[...])

    y = pl.pallas_call(
        body,
        out_shape=jax.ShapeDtypeStruct((B // 128, 128), jnp.float32),
    )(x)
    return y.reshape(B)


def kernel(u_pos, v, u_table, v_table):
    scores = _sc_scores(u_pos, v, u_table, v_table)
    return _logsigmoid_tc(scores)
